# baseline (speedup 1.0000x reference)
import jax
import jax.numpy as jnp
from jax import lax
from jax.experimental import pallas as pl
from jax.experimental.pallas import tpu as pltpu
from jax.experimental.pallas import tpu_sc as plsc

BATCH = 16384
EMBED = 128
LANES = 16

NUM_CORES = 2
NUM_SUBCORES = 16
PAIR = BATCH // NUM_SUBCORES
CHUNK = 64
NBUF = 4
LEAD_CORE = 0
LEAD_N = 544
SIZES_LEAD = [CHUNK] * 8 + [32]
SIZES_LAG = [CHUNK] * 7 + [32]


def _pipeline(base, sizes, uid_hbm, iid_hbm, p_hbm, q_hbm, y_hbm,
              uidx_v, iidx_v, p_rows, q_rows, out_v,
              sem_idx, sem_p, sem_q, sem_out):
  offs = [0]
  for sz in sizes:
    offs.append(offs[-1] + sz)
  total = offs[-1]
  n_chunks = len(sizes)

  first = sizes[0]
  cu0 = pltpu.async_copy(uid_hbm.at[pl.ds(base, first)],
                         uidx_v.at[pl.ds(0, first)], sem_idx)
  ci0 = pltpu.async_copy(iid_hbm.at[pl.ds(base, first)],
                         iidx_v.at[pl.ds(0, first)], sem_idx)
  rest = total - first
  cur = pltpu.async_copy(uid_hbm.at[pl.ds(base + first, rest)],
                         uidx_v.at[pl.ds(first, rest)], sem_idx)
  cir = pltpu.async_copy(iid_hbm.at[pl.ds(base + first, rest)],
                         iidx_v.at[pl.ds(first, rest)], sem_idx)

  def start_gather(k):
    buf = k % NBUF
    sz = sizes[k]
    dst_p = p_rows.at[buf] if sz == CHUNK else p_rows.at[buf].at[pl.ds(0, sz)]
    dst_q = q_rows.at[buf] if sz == CHUNK else q_rows.at[buf].at[pl.ds(0, sz)]
    cp = pltpu.async_copy(p_hbm.at[uidx_v.at[pl.ds(offs[k], sz)]],
                          dst_p, sem_p.at[buf])
    cq = pltpu.async_copy(q_hbm.at[iidx_v.at[pl.ds(offs[k], sz)]],
                          dst_q, sem_q.at[buf])
    return cp, cq

  cu0.wait()
  ci0.wait()
  inflight = {0: start_gather(0)}
  cur.wait()
  cir.wait()
  for k in range(1, NBUF):
    inflight[k] = start_gather(k)
  lane = lax.iota(jnp.int32, LANES)
  out_copies = []

  for k in range(n_chunks):
    cp, cq = inflight.pop(k)
    cp.wait()
    cq.wait()
    buf = k % NBUF

    def group(g, carry, k=k, buf=buf):
      rvec = g * LANES + lane

      def col(d, acc):
        cvec = (jnp.full((LANES,), d, jnp.int32) + lane) & (EMBED - 1)
        pv = plsc.load_gather(p_rows.at[buf], [rvec, cvec])
        qv = plsc.load_gather(q_rows.at[buf], [rvec, cvec])
        return acc + pv * qv

      acc = lax.fori_loop(0, EMBED, col, jnp.zeros((LANES,), jnp.float32),
                          unroll=8)
      out_v[pl.ds(offs[k] + g * LANES, LANES)] = acc
      return carry

    lax.fori_loop(0, sizes[k] // LANES, group, None)

    if k + NBUF < n_chunks:
      inflight[k + NBUF] = start_gather(k + NBUF)

    out_copies.append(pltpu.async_copy(
        out_v.at[pl.ds(offs[k], sizes[k])],
        y_hbm.at[pl.ds(base + offs[k], sizes[k])], sem_out))

  for c in out_copies:
    c.wait()


def _body(uid_hbm, iid_hbm, p_hbm, q_hbm, y_hbm,
          uidx_v, iidx_v, p_rows, q_rows, out_v, sem_idx, sem_p, sem_q,
          sem_out):
  c = lax.axis_index("c")
  s = lax.axis_index("s")
  is_lead = c == LEAD_CORE
  args = (uid_hbm, iid_hbm, p_hbm, q_hbm, y_hbm,
          uidx_v, iidx_v, p_rows, q_rows, out_v,
          sem_idx, sem_p, sem_q, sem_out)

  @pl.when(is_lead)
  def _():
    _pipeline(s * PAIR, SIZES_LEAD, *args)

  @pl.when(jnp.logical_not(is_lead))
  def _():
    _pipeline(s * PAIR + LEAD_N, SIZES_LAG, *args)


@jax.jit
def kernel(user_ids, item_ids, P, Q):
  mesh = plsc.VectorSubcoreMesh(core_axis_name="c", subcore_axis_name="s")
  run = pl.kernel(
      _body,
      out_type=jax.ShapeDtypeStruct((BATCH,), jnp.float32),
      mesh=mesh,
      scratch_types=[
          pltpu.VMEM((LEAD_N,), jnp.int32),
          pltpu.VMEM((LEAD_N,), jnp.int32),
          pltpu.VMEM((NBUF, CHUNK, EMBED), jnp.float32),
          pltpu.VMEM((NBUF, CHUNK, EMBED), jnp.float32),
          pltpu.VMEM((LEAD_N,), jnp.float32),
          pltpu.SemaphoreType.DMA,
          pltpu.SemaphoreType.DMA((NBUF,)),
          pltpu.SemaphoreType.DMA((NBUF,)),
          pltpu.SemaphoreType.DMA,
      ],
      compiler_params=pltpu.CompilerParams(
          needs_layout_passes=False,
          skip_device_barrier=True,
          disable_bounds_checks=True,
      ),
  )
  return run(user_ids, item_ids, P, Q)


# speedup vs baseline: 1.0146x; 1.0146x over previous
import jax
import jax.numpy as jnp
from jax import lax
from jax.experimental import pallas as pl
from jax.experimental.pallas import tpu as pltpu
from jax.experimental.pallas import tpu_sc as plsc

BATCH = 16384
EMBED = 128
LANES = 16

NUM_CORES = 2
NUM_SUBCORES = 16
PAIR = BATCH // NUM_SUBCORES
CHUNK = 64
NBUF = 4
LEAD_CORE = 1
LEAD_N = 544
SIZES_LEAD = [CHUNK] * 8 + [32]
SIZES_LAG = [CHUNK] * 7 + [32]


def _pipeline(base, sizes, uid_hbm, iid_hbm, p_hbm, q_hbm, y_hbm,
              uidx_v, iidx_v, p_rows, q_rows, out_v,
              sem_idx, sem_p, sem_q, sem_out):
  offs = [0]
  for sz in sizes:
    offs.append(offs[-1] + sz)
  total = offs[-1]
  n_chunks = len(sizes)

  first = sizes[0]
  cu0 = pltpu.async_copy(uid_hbm.at[pl.ds(base, first)],
                         uidx_v.at[pl.ds(0, first)], sem_idx)
  ci0 = pltpu.async_copy(iid_hbm.at[pl.ds(base, first)],
                         iidx_v.at[pl.ds(0, first)], sem_idx)
  rest = total - first
  cur = pltpu.async_copy(uid_hbm.at[pl.ds(base + first, rest)],
                         uidx_v.at[pl.ds(first, rest)], sem_idx)
  cir = pltpu.async_copy(iid_hbm.at[pl.ds(base + first, rest)],
                         iidx_v.at[pl.ds(first, rest)], sem_idx)

  def start_gather(k):
    buf = k % NBUF
    sz = sizes[k]
    dst_p = p_rows.at[buf] if sz == CHUNK else p_rows.at[buf].at[pl.ds(0, sz)]
    dst_q = q_rows.at[buf] if sz == CHUNK else q_rows.at[buf].at[pl.ds(0, sz)]
    cp = pltpu.async_copy(p_hbm.at[uidx_v.at[pl.ds(offs[k], sz)]],
                          dst_p, sem_p.at[buf])
    cq = pltpu.async_copy(q_hbm.at[iidx_v.at[pl.ds(offs[k], sz)]],
                          dst_q, sem_q.at[buf])
    return cp, cq

  cu0.wait()
  ci0.wait()
  inflight = {0: start_gather(0)}
  cur.wait()
  cir.wait()
  for k in range(1, NBUF):
    inflight[k] = start_gather(k)
  lane = lax.iota(jnp.int32, LANES)
  out_copies = []

  for k in range(n_chunks):
    cp, cq = inflight.pop(k)
    cp.wait()
    cq.wait()
    buf = k % NBUF

    def group(g, carry, k=k, buf=buf):
      rvec = g * LANES + lane

      def col(d, acc):
        cvec = (jnp.full((LANES,), d, jnp.int32) + lane) & (EMBED - 1)
        pv = plsc.load_gather(p_rows.at[buf], [rvec, cvec])
        qv = plsc.load_gather(q_rows.at[buf], [rvec, cvec])
        return acc + pv * qv

      acc = lax.fori_loop(0, EMBED, col, jnp.zeros((LANES,), jnp.float32),
                          unroll=8)
      out_v[pl.ds(offs[k] + g * LANES, LANES)] = acc
      return carry

    lax.fori_loop(0, sizes[k] // LANES, group, None)

    if k + NBUF < n_chunks:
      inflight[k + NBUF] = start_gather(k + NBUF)

    out_copies.append(pltpu.async_copy(
        out_v.at[pl.ds(offs[k], sizes[k])],
        y_hbm.at[pl.ds(base + offs[k], sizes[k])], sem_out))

  for c in out_copies:
    c.wait()


def _body(uid_hbm, iid_hbm, p_hbm, q_hbm, y_hbm,
          uidx_v, iidx_v, p_rows, q_rows, out_v, sem_idx, sem_p, sem_q,
          sem_out):
  c = lax.axis_index("c")
  s = lax.axis_index("s")
  is_lead = c == LEAD_CORE
  args = (uid_hbm, iid_hbm, p_hbm, q_hbm, y_hbm,
          uidx_v, iidx_v, p_rows, q_rows, out_v,
          sem_idx, sem_p, sem_q, sem_out)

  @pl.when(is_lead)
  def _():
    _pipeline(s * PAIR, SIZES_LEAD, *args)

  @pl.when(jnp.logical_not(is_lead))
  def _():
    _pipeline(s * PAIR + LEAD_N, SIZES_LAG, *args)


@jax.jit
def kernel(user_ids, item_ids, P, Q):
  mesh = plsc.VectorSubcoreMesh(core_axis_name="c", subcore_axis_name="s")
  run = pl.kernel(
      _body,
      out_type=jax.ShapeDtypeStruct((BATCH,), jnp.float32),
      mesh=mesh,
      scratch_types=[
          pltpu.VMEM((LEAD_N,), jnp.int32),
          pltpu.VMEM((LEAD_N,), jnp.int32),
          pltpu.VMEM((NBUF, CHUNK, EMBED), jnp.float32),
          pltpu.VMEM((NBUF, CHUNK, EMBED), jnp.float32),
          pltpu.VMEM((LEAD_N,), jnp.float32),
          pltpu.SemaphoreType.DMA,
          pltpu.SemaphoreType.DMA((NBUF,)),
          pltpu.SemaphoreType.DMA((NBUF,)),
          pltpu.SemaphoreType.DMA,
      ],
      compiler_params=pltpu.CompilerParams(
          needs_layout_passes=False,
          skip_device_barrier=True,
          disable_bounds_checks=True,
      ),
  )
  return run(user_ids, item_ids, P, Q)


# speedup vs baseline: 1.0602x; 1.0450x over previous
import jax
import jax.numpy as jnp
from jax import lax
from jax.experimental import pallas as pl
from jax.experimental.pallas import tpu as pltpu
from jax.experimental.pallas import tpu_sc as plsc

BATCH = 16384
EMBED = 128
LANES = 16

NUM_CORES = 2
NUM_SUBCORES = 16
NUM_WORKERS = NUM_CORES * NUM_SUBCORES
B_PER_W = BATCH // NUM_WORKERS
CHUNK = 64
N_CHUNKS = B_PER_W // CHUNK
NBUF = 4


def _body(uid_hbm, iid_hbm, p_hbm, q_hbm, y_hbm,
          uidx_v, iidx_v, p_rows, q_rows, out_v, sem_idx, sem_p, sem_q,
          sem_out):
  wid = lax.axis_index("s") * NUM_CORES + lax.axis_index("c")
  base = wid * B_PER_W

  cu0 = pltpu.async_copy(uid_hbm.at[pl.ds(base, CHUNK)],
                         uidx_v.at[pl.ds(0, CHUNK)], sem_idx)
  ci0 = pltpu.async_copy(iid_hbm.at[pl.ds(base, CHUNK)],
                         iidx_v.at[pl.ds(0, CHUNK)], sem_idx)
  REST = B_PER_W - CHUNK
  cur = pltpu.async_copy(uid_hbm.at[pl.ds(base + CHUNK, REST)],
                         uidx_v.at[pl.ds(CHUNK, REST)], sem_idx)
  cir = pltpu.async_copy(iid_hbm.at[pl.ds(base + CHUNK, REST)],
                         iidx_v.at[pl.ds(CHUNK, REST)], sem_idx)

  def start_gather(k):
    buf = k % NBUF
    cp = pltpu.async_copy(p_hbm.at[uidx_v.at[pl.ds(k * CHUNK, CHUNK)]],
                          p_rows.at[buf], sem_p.at[buf])
    cq = pltpu.async_copy(q_hbm.at[iidx_v.at[pl.ds(k * CHUNK, CHUNK)]],
                          q_rows.at[buf], sem_q.at[buf])
    return cp, cq

  cu0.wait()
  ci0.wait()
  inflight = {0: start_gather(0)}
  cur.wait()
  cir.wait()
  for k in range(1, NBUF):
    inflight[k] = start_gather(k)
  lane = lax.iota(jnp.int32, LANES)
  out_copies = []

  for k in range(N_CHUNKS):
    cp, cq = inflight.pop(k)
    cp.wait()
    cq.wait()
    buf = k % NBUF

    def group(g, carry, k=k, buf=buf):
      rvec = g * LANES + lane

      def col(d, acc):
        cvec = (jnp.full((LANES,), d, jnp.int32) + lane) & (EMBED - 1)
        pv = plsc.load_gather(p_rows.at[buf], [rvec, cvec])
        qv = plsc.load_gather(q_rows.at[buf], [rvec, cvec])
        return acc + pv * qv

      acc = lax.fori_loop(0, EMBED, col, jnp.zeros((LANES,), jnp.float32),
                          unroll=8)
      out_v[pl.ds(k * CHUNK + g * LANES, LANES)] = acc
      return carry

    lax.fori_loop(0, CHUNK // LANES, group, None)

    if k + NBUF < N_CHUNKS:
      inflight[k + NBUF] = start_gather(k + NBUF)

    out_copies.append(pltpu.async_copy(
        out_v.at[pl.ds(k * CHUNK, CHUNK)],
        y_hbm.at[pl.ds(base + k * CHUNK, CHUNK)], sem_out))

  for c in out_copies:
    c.wait()


@jax.jit
def kernel(user_ids, item_ids, P, Q):
  mesh = plsc.VectorSubcoreMesh(core_axis_name="c", subcore_axis_name="s")
  run = pl.kernel(
      _body,
      out_type=jax.ShapeDtypeStruct((BATCH,), jnp.float32),
      mesh=mesh,
      scratch_types=[
          pltpu.VMEM((B_PER_W,), jnp.int32),
          pltpu.VMEM((B_PER_W,), jnp.int32),
          pltpu.VMEM((NBUF, CHUNK, EMBED), jnp.float32),
          pltpu.VMEM((NBUF, CHUNK, EMBED), jnp.float32),
          pltpu.VMEM((B_PER_W,), jnp.float32),
          pltpu.SemaphoreType.DMA,
          pltpu.SemaphoreType.DMA((NBUF,)),
          pltpu.SemaphoreType.DMA((NBUF,)),
          pltpu.SemaphoreType.DMA,
      ],
      compiler_params=pltpu.CompilerParams(
          needs_layout_passes=False,
          skip_device_barrier=True,
          disable_bounds_checks=True,
      ),
  )
  return run(user_ids, item_ids, P, Q)


# speedup vs baseline: 1.0633x; 1.0029x over previous
import jax
import jax.numpy as jnp
from jax import lax
from jax.experimental import pallas as pl
from jax.experimental.pallas import tpu as pltpu
from jax.experimental.pallas import tpu_sc as plsc

BATCH = 16384
EMBED = 128
LANES = 16

NUM_CORES = 2
NUM_SUBCORES = 16
NUM_WORKERS = NUM_CORES * NUM_SUBCORES
B_PER_W = BATCH // NUM_WORKERS
CHUNK = 64
N_CHUNKS = B_PER_W // CHUNK
NBUF = 4


def _body(uid_hbm, iid_hbm, p_hbm, q_hbm, y_hbm,
          uidx_v, iidx_v, p_rows, q_rows, out_v, sem_idx, sem_p, sem_q,
          sem_out):
  wid = lax.axis_index("s") * NUM_CORES + lax.axis_index("c")
  base = wid * B_PER_W

  cu0 = pltpu.async_copy(uid_hbm.at[pl.ds(base, CHUNK)],
                         uidx_v.at[pl.ds(0, CHUNK)], sem_idx)
  ci0 = pltpu.async_copy(iid_hbm.at[pl.ds(base, CHUNK)],
                         iidx_v.at[pl.ds(0, CHUNK)], sem_idx)
  REST = B_PER_W - CHUNK
  cur = pltpu.async_copy(uid_hbm.at[pl.ds(base + CHUNK, REST)],
                         uidx_v.at[pl.ds(CHUNK, REST)], sem_idx)
  cir = pltpu.async_copy(iid_hbm.at[pl.ds(base + CHUNK, REST)],
                         iidx_v.at[pl.ds(CHUNK, REST)], sem_idx)

  def start_gather(k):
    buf = k % NBUF
    cp = pltpu.async_copy(p_hbm.at[uidx_v.at[pl.ds(k * CHUNK, CHUNK)]],
                          p_rows.at[buf], sem_p.at[buf])
    cq = pltpu.async_copy(q_hbm.at[iidx_v.at[pl.ds(k * CHUNK, CHUNK)]],
                          q_rows.at[buf], sem_q.at[buf])
    return cp, cq

  cu0.wait()
  ci0.wait()
  inflight = {0: start_gather(0)}
  cur.wait()
  cir.wait()
  for k in range(1, NBUF):
    inflight[k] = start_gather(k)
  lane = lax.iota(jnp.int32, LANES)
  out_copies = []

  for k in range(N_CHUNKS):
    cp, cq = inflight.pop(k)
    cp.wait()
    cq.wait()
    buf = k % NBUF

    def group(g, carry, k=k, buf=buf):
      rvec = g * LANES + lane

      def col(d, acc):
        cvec = (jnp.full((LANES,), d, jnp.int32) + lane) & (EMBED - 1)
        pv = plsc.load_gather(p_rows.at[buf], [rvec, cvec])
        qv = plsc.load_gather(q_rows.at[buf], [rvec, cvec])
        return acc + pv * qv

      acc = lax.fori_loop(0, EMBED, col, jnp.zeros((LANES,), jnp.float32),
                          unroll=8)
      out_v[pl.ds(k * CHUNK + g * LANES, LANES)] = acc
      return carry

    lax.fori_loop(0, CHUNK // LANES, group, None)

    if k + NBUF < N_CHUNKS:
      inflight[k + NBUF] = start_gather(k + NBUF)

    out_copies.append(pltpu.async_copy(
        out_v.at[pl.ds(k * CHUNK, CHUNK)],
        y_hbm.at[pl.ds(base + k * CHUNK, CHUNK)], sem_out))

  for c in out_copies:
    c.wait()


@jax.jit
def kernel(user_ids, item_ids, P, Q):
  user_ids = user_ids.astype(jnp.int32)
  item_ids = item_ids.astype(jnp.int32)
  mesh = plsc.VectorSubcoreMesh(core_axis_name="c", subcore_axis_name="s")
  run = pl.kernel(
      _body,
      out_type=jax.ShapeDtypeStruct((BATCH,), jnp.float32),
      mesh=mesh,
      scratch_types=[
          pltpu.VMEM((B_PER_W,), jnp.int32),
          pltpu.VMEM((B_PER_W,), jnp.int32),
          pltpu.VMEM((NBUF, CHUNK, EMBED), jnp.float32),
          pltpu.VMEM((NBUF, CHUNK, EMBED), jnp.float32),
          pltpu.VMEM((B_PER_W,), jnp.float32),
          pltpu.SemaphoreType.DMA,
          pltpu.SemaphoreType.DMA((NBUF,)),
          pltpu.SemaphoreType.DMA((NBUF,)),
          pltpu.SemaphoreType.DMA,
      ],
      compiler_params=pltpu.CompilerParams(
          needs_layout_passes=False,
          skip_device_barrier=True,
          disable_bounds_checks=True,
      ),
  )
  return run(user_ids, item_ids, P, Q)
